# baseline (speedup 1.0000x reference)
import functools

import jax
import jax.numpy as jnp
from jax.experimental import pallas as pl
from jax.experimental.pallas import tpu as pltpu

_LANE = 128
_TILE_B = 2048
_VMEM_LIMIT = 64 * 1024 * 1024


def _matmul_bias(x, w, b):
    logits = jax.lax.dot_general(
        x, w, dimension_numbers=(((1,), (1,)), ((), ())),
        preferred_element_type=jnp.float32)
    return logits + b


def _ring_body(x_hbm, w_ref, b_ref, o_hbm, x_buf, o_buf, in_sem, out_sem,
               *, tile, nb):
    c = pl.program_id(0)
    base = c * (nb * tile)

    def dma_in(slot, step):
        return pltpu.make_async_copy(
            x_hbm.at[pl.ds(base + step * tile, tile), :],
            x_buf.at[slot], in_sem.at[slot])

    def dma_out(slot, step):
        return pltpu.make_async_copy(
            o_buf.at[slot],
            o_hbm.at[pl.ds(base + step * tile, tile), :], out_sem.at[slot])

    dma_in(0, 0).start()
    dma_in(1, 1).start()

    def body(step, _):
        cur = jax.lax.rem(step, 3)
        oslot = jax.lax.rem(step, 2)

        @pl.when(step + 2 < nb)
        def _():
            dma_in(jax.lax.rem(step + 2, 3), step + 2).start()

        dma_in(cur, step).wait()

        @pl.when(step >= 2)
        def _():
            dma_out(oslot, step - 2).wait()

        o_buf[oslot] = _matmul_bias(
            x_buf[cur], w_ref[...], b_ref[...]).astype(o_buf.dtype)
        dma_out(oslot, step).start()
        return ()

    jax.lax.fori_loop(0, nb, body, (), unroll=False)
    dma_out(jax.lax.rem(nb - 2, 2), nb - 2).wait()
    dma_out(jax.lax.rem(nb - 1, 2), nb - 1).wait()


def _flat_body(x_ref, w_ref, b_ref, o_ref):
    n = o_ref.shape[-1]
    o_ref[...] = _matmul_bias(
        x_ref[...], w_ref[...], b_ref[...])[:, :n].astype(o_ref.dtype)


def _pick_tile(B):
    if B <= _TILE_B:
        return B
    t = _TILE_B
    while B % t and t > 8:
        t //= 2
    return t


@jax.jit
def kernel(pooled_output, weight, bias):
    B, H = pooled_output.shape
    L = weight.shape[0]

    Lp = pl.cdiv(L, _LANE) * _LANE
    w_p = weight
    bias_p = bias
    if Lp != L:
        w_p = jnp.pad(weight, ((0, Lp - L), (0, 0)))
        bias_p = jnp.pad(bias, (0, Lp - L))
    b2 = bias_p.reshape(1, Lp)

    tile_b = _pick_tile(B)
    cost = pl.CostEstimate(
        flops=2 * B * H * Lp,
        transcendentals=0,
        bytes_accessed=B * H * 4 + Lp * H * 4 + B * L * 4)
    out_shape = jax.ShapeDtypeStruct((B, L), pooled_output.dtype)

    nb = B // (2 * tile_b)
    if L == Lp and nb >= 4 and 2 * nb * tile_b == B:
        return pl.pallas_call(
            functools.partial(_ring_body, tile=tile_b, nb=nb),
            grid=(2,),
            in_specs=[
                pl.BlockSpec(memory_space=pl.ANY),
                pl.BlockSpec((Lp, H), lambda c: (0, 0)),
                pl.BlockSpec((1, Lp), lambda c: (0, 0)),
            ],
            out_specs=pl.BlockSpec(memory_space=pl.ANY),
            out_shape=out_shape,
            scratch_shapes=[
                pltpu.VMEM((3, tile_b, H), pooled_output.dtype),
                pltpu.VMEM((2, tile_b, L), pooled_output.dtype),
                pltpu.SemaphoreType.DMA((3,)),
                pltpu.SemaphoreType.DMA((2,)),
            ],
            compiler_params=pltpu.CompilerParams(
                dimension_semantics=("parallel",),
                vmem_limit_bytes=_VMEM_LIMIT),
            cost_estimate=cost,
        )(pooled_output, w_p, b2)

    return pl.pallas_call(
        _flat_body,
        grid=(pl.cdiv(B, tile_b),),
        in_specs=[
            pl.BlockSpec((tile_b, H), lambda i: (i, 0)),
            pl.BlockSpec((Lp, H), lambda i: (0, 0)),
            pl.BlockSpec((1, Lp), lambda i: (0, 0)),
        ],
        out_specs=pl.BlockSpec((tile_b, L), lambda i: (i, 0)),
        out_shape=out_shape,
        compiler_params=pltpu.CompilerParams(
            dimension_semantics=("parallel",),
            vmem_limit_bytes=_VMEM_LIMIT),
        cost_estimate=cost,
    )(pooled_output, w_p, b2)


# speedup vs baseline: 1.0744x; 1.0744x over previous
import functools

import jax
import jax.numpy as jnp
from jax.experimental import pallas as pl
from jax.experimental.pallas import tpu as pltpu

_LANE = 128
_TILE_B = 4096
_VMEM_LIMIT = 64 * 1024 * 1024


def _head_body(x_ref, w_ref, b_ref, o_ref):
    logits = jax.lax.dot_general(
        x_ref[...], w_ref[...],
        dimension_numbers=(((1,), (1,)), ((), ())),
        preferred_element_type=jnp.float32)
    n = o_ref.shape[-1]
    o_ref[...] = (logits + b_ref[...])[:, :n].astype(o_ref.dtype)


def _pick_tile(B):
    if B <= _TILE_B:
        return B
    t = _TILE_B
    while B % t and t > 8:
        t //= 2
    return t


@jax.jit
def kernel(pooled_output, weight, bias):
    B, H = pooled_output.shape
    L = weight.shape[0]

    Lp = pl.cdiv(L, _LANE) * _LANE
    w_p = weight
    bias_p = bias
    if Lp != L:
        w_p = jnp.pad(weight, ((0, Lp - L), (0, 0)))
        bias_p = jnp.pad(bias, (0, Lp - L))
    b2 = bias_p.reshape(1, Lp)

    tile_b = _pick_tile(B)

    return pl.pallas_call(
        _head_body,
        grid=(pl.cdiv(B, tile_b),),
        in_specs=[
            pl.BlockSpec((tile_b, H), lambda i: (i, 0)),
            pl.BlockSpec((Lp, H), lambda i: (0, 0)),
            pl.BlockSpec((1, Lp), lambda i: (0, 0)),
        ],
        out_specs=pl.BlockSpec((tile_b, L), lambda i: (i, 0)),
        out_shape=jax.ShapeDtypeStruct((B, L), pooled_output.dtype),
        compiler_params=pltpu.CompilerParams(
            dimension_semantics=("parallel",),
            vmem_limit_bytes=_VMEM_LIMIT),
        cost_estimate=pl.CostEstimate(
            flops=2 * B * H * Lp,
            transcendentals=0,
            bytes_accessed=B * H * 4 + Lp * H * 4 + B * L * 4),
    )(pooled_output, w_p, b2)


# speedup vs baseline: 1.0789x; 1.0042x over previous
import functools

import jax
import jax.numpy as jnp
from jax.experimental import pallas as pl
from jax.experimental.pallas import tpu as pltpu

_LANE = 128
_TILE_B = 4096
_VMEM_LIMIT = 64 * 1024 * 1024


def _head_body(x_ref, w_ref, b_ref, o_ref):
    logits = jax.lax.dot_general(
        x_ref[...].astype(jnp.bfloat16), w_ref[...].astype(jnp.bfloat16),
        dimension_numbers=(((1,), (1,)), ((), ())),
        preferred_element_type=jnp.float32)
    n = o_ref.shape[-1]
    o_ref[...] = (logits + b_ref[...])[:, :n].astype(o_ref.dtype)


def _pick_tile(B):
    if B <= _TILE_B:
        return B
    t = _TILE_B
    while B % t and t > 8:
        t //= 2
    return t


@jax.jit
def kernel(pooled_output, weight, bias):
    B, H = pooled_output.shape
    L = weight.shape[0]

    Lp = pl.cdiv(L, _LANE) * _LANE
    w_p = weight
    bias_p = bias
    if Lp != L:
        w_p = jnp.pad(weight, ((0, Lp - L), (0, 0)))
        bias_p = jnp.pad(bias, (0, Lp - L))
    b2 = bias_p.reshape(1, Lp)

    tile_b = _pick_tile(B)

    return pl.pallas_call(
        _head_body,
        grid=(pl.cdiv(B, tile_b),),
        in_specs=[
            pl.BlockSpec((tile_b, H), lambda i: (i, 0)),
            pl.BlockSpec((Lp, H), lambda i: (0, 0)),
            pl.BlockSpec((1, Lp), lambda i: (0, 0)),
        ],
        out_specs=pl.BlockSpec((tile_b, L), lambda i: (i, 0)),
        out_shape=jax.ShapeDtypeStruct((B, L), pooled_output.dtype),
        compiler_params=pltpu.CompilerParams(
            dimension_semantics=("parallel",),
            vmem_limit_bytes=_VMEM_LIMIT),
        cost_estimate=pl.CostEstimate(
            flops=2 * B * H * Lp,
            transcendentals=0,
            bytes_accessed=B * H * 4 + Lp * H * 4 + B * L * 4),
    )(pooled_output, w_p, b2)
